# Initial kernel scaffold; baseline (speedup 1.0000x reference)
import jax
import jax.numpy as jnp
from jax.experimental import pallas as pl


def kernel(white, black, psqt_w, acc_w, acc_b, layer_w):
    raise NotImplementedError("write your pallas kernel here")


# speedup vs baseline: 1.7847x; 1.7847x over previous
import jax
import jax.numpy as jnp
from jax.experimental import pallas as pl
from jax.experimental.pallas import tpu as pltpu

BM = 1024
BK = 2048


def _nnue_body(w_ref, b_ref, awt_ref, pw_ref, bias_ref, lwt_ref, out_ref,
               accw_s, accb_s, psqt_s):
    k = pl.program_id(1)
    nk = pl.num_programs(1)

    wt = w_ref[...]
    bt = b_ref[...]

    diff = wt - bt
    p0 = pw_ref[0:1, :]
    p1 = pw_ref[1:2, :]
    ps0 = jnp.sum(diff * p0, axis=1, keepdims=True)
    ps1 = jnp.sum(diff * p1, axis=1, keepdims=True)
    psqt_part = jnp.concatenate([ps0, ps1], axis=1)

    awt = awt_ref[...]
    dw = jnp.dot(wt, awt, preferred_element_type=jnp.float32)
    db = jnp.dot(bt, awt, preferred_element_type=jnp.float32)

    @pl.when(k == 0)
    def _init():
        accw_s[...] = dw
        accb_s[...] = db
        psqt_s[...] = psqt_part

    @pl.when(k > 0)
    def _acc():
        accw_s[...] += dw
        accb_s[...] += db
        psqt_s[...] += psqt_part

    @pl.when(k == nk - 1)
    def _fin():
        bias = bias_ref[...]
        cw = jnp.clip(accw_s[...] + bias, 0.0, 1.0)
        cb = jnp.clip(accb_s[...] + bias, 0.0, 1.0)
        pos = jnp.dot(cw - cb, lwt_ref[...], preferred_element_type=jnp.float32)
        out_ref[...] = psqt_s[...] + pos


def kernel(white, black, psqt_w, acc_w, acc_b, layer_w):
    m, kdim = white.shape
    nacc = acc_w.shape[0]
    bm = min(BM, m)
    bk = min(BK, kdim)
    grid = (m // bm, kdim // bk)

    awt = acc_w.T
    lwt = layer_w.T
    bias = acc_b.reshape(1, nacc)

    return pl.pallas_call(
        _nnue_body,
        grid=grid,
        in_specs=[
            pl.BlockSpec((bm, bk), lambda i, j: (i, j)),
            pl.BlockSpec((bm, bk), lambda i, j: (i, j)),
            pl.BlockSpec((bk, nacc), lambda i, j: (j, 0)),
            pl.BlockSpec((2, bk), lambda i, j: (0, j)),
            pl.BlockSpec((1, nacc), lambda i, j: (0, 0)),
            pl.BlockSpec((nacc, 2), lambda i, j: (0, 0)),
        ],
        out_specs=pl.BlockSpec((bm, 2), lambda i, j: (i, 0)),
        out_shape=jax.ShapeDtypeStruct((m, 2), jnp.float32),
        scratch_shapes=[
            pltpu.VMEM((bm, nacc), jnp.float32),
            pltpu.VMEM((bm, nacc), jnp.float32),
            pltpu.VMEM((bm, 2), jnp.float32),
        ],
        compiler_params=pltpu.CompilerParams(
            dimension_semantics=("parallel", "arbitrary"),
        ),
    )(white, black, awt, psqt_w, bias, lwt)


# speedup vs baseline: 1.8202x; 1.0199x over previous
import jax
import jax.numpy as jnp
from jax.experimental import pallas as pl
from jax.experimental.pallas import tpu as pltpu

BM = 1024
BK = 2048


def _nnue_body(w_ref, b_ref, awt_ref, pw_ref, bias_ref, lwt_ref, out_ref,
               accw_s, accb_s, psqt_s):
    k = pl.program_id(1)
    nk = pl.num_programs(1)

    wt = w_ref[...]
    bt = b_ref[...]

    diff = wt - bt
    p0 = pw_ref[0:1, :]
    p1 = pw_ref[1:2, :]
    ps0 = jnp.sum(diff * p0, axis=1, keepdims=True)
    ps1 = jnp.sum(diff * p1, axis=1, keepdims=True)
    psqt_part = jnp.concatenate([ps0, ps1], axis=1)

    awt = awt_ref[...]
    dw = jnp.dot(wt.astype(jnp.bfloat16), awt, preferred_element_type=jnp.float32)
    db = jnp.dot(bt.astype(jnp.bfloat16), awt, preferred_element_type=jnp.float32)

    @pl.when(k == 0)
    def _init():
        accw_s[...] = dw
        accb_s[...] = db
        psqt_s[...] = psqt_part

    @pl.when(k > 0)
    def _acc():
        accw_s[...] += dw
        accb_s[...] += db
        psqt_s[...] += psqt_part

    @pl.when(k == nk - 1)
    def _fin():
        bias = bias_ref[...]
        cw = jnp.clip(accw_s[...] + bias, 0.0, 1.0)
        cb = jnp.clip(accb_s[...] + bias, 0.0, 1.0)
        pos = jnp.dot(cw - cb, lwt_ref[...], preferred_element_type=jnp.float32)
        out_ref[...] = psqt_s[...] + pos


def kernel(white, black, psqt_w, acc_w, acc_b, layer_w):
    m, kdim = white.shape
    nacc = acc_w.shape[0]
    bm = min(BM, m)
    bk = min(BK, kdim)
    grid = (m // bm, kdim // bk)

    awt = acc_w.T.astype(jnp.bfloat16)
    lwt = layer_w.T
    bias = acc_b.reshape(1, nacc)

    return pl.pallas_call(
        _nnue_body,
        grid=grid,
        in_specs=[
            pl.BlockSpec((bm, bk), lambda i, j: (i, j)),
            pl.BlockSpec((bm, bk), lambda i, j: (i, j)),
            pl.BlockSpec((bk, nacc), lambda i, j: (j, 0)),
            pl.BlockSpec((2, bk), lambda i, j: (0, j)),
            pl.BlockSpec((1, nacc), lambda i, j: (0, 0)),
            pl.BlockSpec((nacc, 2), lambda i, j: (0, 0)),
        ],
        out_specs=pl.BlockSpec((bm, 2), lambda i, j: (i, 0)),
        out_shape=jax.ShapeDtypeStruct((m, 2), jnp.float32),
        scratch_shapes=[
            pltpu.VMEM((bm, nacc), jnp.float32),
            pltpu.VMEM((bm, nacc), jnp.float32),
            pltpu.VMEM((bm, 2), jnp.float32),
        ],
        compiler_params=pltpu.CompilerParams(
            dimension_semantics=("parallel", "arbitrary"),
        ),
    )(white, black, awt, psqt_w, bias, lwt)
